# baseline (device time: 316148 ns/iter reference)
import jax
import jax.numpy as jnp
from jax import lax
from jax.experimental import pallas as pl
from jax.experimental.pallas import tpu as pltpu

N_DEV = 4
NB = 8


def kernel(x, w_mat):
    m, k = x.shape
    _, n = w_mat.shape
    m_out = m // N_DEV
    nb = n // NB
    nh = nb // 2
    nq = nh // 2
    f32 = jnp.float32
    bf16 = jnp.bfloat16

    def body(
        x_ref, w_ref, out_ref,
        own_part, bd_part, send_cw, send_ccw, recv_cw, recv_ccw,
        send_sems_cw, send_sems_ccw, recv_sems_cw, recv_sems_ccw,
        credit_cw, credit_ccw,
    ):
        j = pl.program_id(0)
        my = lax.axis_index("i")
        left = lax.rem(my + (N_DEV - 1), N_DEV)
        right = lax.rem(my + 1, N_DEV)

        cw = (send_cw, recv_cw, send_sems_cw, recv_sems_cw,
              credit_cw, left, right)
        ccw = (send_ccw, recv_ccw, send_sems_ccw, recv_sems_ccw,
               credit_ccw, right, left)

        def piece_rdma(d, s, p, target):
            sbuf, rbuf, ssems, rsems = d[0], d[1], d[2], d[3]
            return pltpu.make_async_remote_copy(
                src_ref=sbuf.at[:, p * nq:(p + 1) * nq],
                dst_ref=rbuf.at[s, :, p * nq:(p + 1) * nq],
                send_sem=ssems.at[p],
                recv_sem=rsems.at[s, p],
                device_id=(target,),
                device_id_type=pl.DeviceIdType.MESH,
            )

        def xslice(off):
            c = lax.rem(my + off, N_DEV)
            return x_ref[pl.ds(c * m_out, m_out), :].astype(bf16)

        def w_lo():
            return w_ref[:, :nh].astype(bf16)

        def w_hi():
            return w_ref[:, nh:].astype(bf16)

        def wait_send(d, p):
            piece_rdma(d, 0, p, d[6]).wait_send()

        def wait_all_sends():
            for d in (cw, ccw):
                for p in (0, 1):
                    wait_send(d, p)

        def wait_credits():
            pl.semaphore_wait(credit_cw, 2)
            pl.semaphore_wait(credit_ccw, 2)

        def start_piece(d, s, p):
            piece_rdma(d, s, p, d[6]).start()

        def recv_piece(d, s, p):
            piece_rdma(d, s, p, d[5]).wait_recv()

        def grant_credits(inc):
            pl.semaphore_signal(
                credit_cw, inc=inc,
                device_id=(left,), device_id_type=pl.DeviceIdType.MESH,
            )
            pl.semaphore_signal(
                credit_ccw, inc=inc,
                device_id=(right,), device_id_type=pl.DeviceIdType.MESH,
            )

        def hop(s_recv, s_send):
            for p in (0, 1):
                for d, off in ((cw, 0), (ccw, nh)):
                    lo, hi = off + p * nq, off + (p + 1) * nq
                    recv_piece(d, s_recv, p)
                    acc = (bd_part[:, lo:hi].astype(f32)
                           + d[1][s_recv, :, p * nq:(p + 1) * nq].astype(f32))
                    wait_send(d, p)
                    d[0][:, p * nq:(p + 1) * nq] = acc.astype(bf16)
                    start_piece(d, s_send, p)

        def finalize():
            for p in (0, 1):
                recv_piece(cw, 2, p)
                recv_piece(ccw, 2, p)
            out_ref[:, :nh] = jnp.maximum(
                own_part[:, :nh].astype(f32) + recv_cw[2, :, :].astype(f32), 0.0)
            out_ref[:, nh:] = jnp.maximum(
                own_part[:, nh:].astype(f32) + recv_ccw[2, :, :].astype(f32), 0.0)

        @pl.when(j == 0)
        def _():
            barrier_sem = pltpu.get_barrier_semaphore()
            for nbr in (left, right):
                pl.semaphore_signal(
                    barrier_sem, inc=1,
                    device_id=(nbr,), device_id_type=pl.DeviceIdType.MESH,
                )
            pl.semaphore_wait(barrier_sem, 2)

        @pl.when(j < NB)
        def _():
            a_cw = jnp.dot(xslice(3), w_lo(), preferred_element_type=f32)
            a_ccw = jnp.dot(xslice(1), w_hi(), preferred_element_type=f32)

            @pl.when(j > 0)
            def _():
                wait_all_sends()
                wait_credits()
            send_cw[...] = a_cw.astype(bf16)
            send_ccw[...] = a_ccw.astype(bf16)
            for p in (0, 1):
                start_piece(cw, 0, p)
                start_piece(ccw, 0, p)

            bd_part[:, :nh] = jnp.dot(
                xslice(2), w_lo(), preferred_element_type=f32).astype(bf16)
            bd_part[:, nh:] = jnp.dot(
                xslice(2), w_hi(), preferred_element_type=f32).astype(bf16)

            @pl.when(j > 0)
            def _():
                finalize()
                grant_credits(2)

            @pl.when(j > 0)
            def _():
                wait_credits()
            hop(0, 1)

            @pl.when(j < NB - 1)
            def _():
                grant_credits(2)

            bd_part[:, :nh] = jnp.dot(
                xslice(1), w_lo(), preferred_element_type=f32).astype(bf16)
            bd_part[:, nh:] = jnp.dot(
                xslice(3), w_hi(), preferred_element_type=f32).astype(bf16)

            own_part[:, :nh] = jnp.dot(
                xslice(0), w_lo(), preferred_element_type=f32).astype(bf16)
            own_part[:, nh:] = jnp.dot(
                xslice(0), w_hi(), preferred_element_type=f32).astype(bf16)

            @pl.when(j > 0)
            def _():
                wait_credits()
            hop(1, 2)

            @pl.when(j < NB - 1)
            def _():
                grant_credits(2)

        @pl.when(j == NB)
        def _():
            finalize()
            wait_all_sends()

    return pl.pallas_call(
        body,
        grid=(NB + 1,),
        out_shape=jax.ShapeDtypeStruct((m_out, n), f32),
        in_specs=[
            pl.BlockSpec((m, k), lambda j: (0, 0)),
            pl.BlockSpec((k, nb), lambda j: (0, jnp.minimum(j, NB - 1))),
        ],
        out_specs=pl.BlockSpec((m_out, nb), lambda j: (0, jnp.maximum(j - 1, 0))),
        scratch_shapes=[
            pltpu.VMEM((m_out, nb), bf16),
            pltpu.VMEM((m_out, nb), bf16),
            pltpu.VMEM((m_out, nh), bf16),
            pltpu.VMEM((m_out, nh), bf16),
            pltpu.VMEM((N_DEV - 1, m_out, nh), bf16),
            pltpu.VMEM((N_DEV - 1, m_out, nh), bf16),
            pltpu.SemaphoreType.DMA((2,)),
            pltpu.SemaphoreType.DMA((2,)),
            pltpu.SemaphoreType.DMA((N_DEV - 1, 2)),
            pltpu.SemaphoreType.DMA((N_DEV - 1, 2)),
            pltpu.SemaphoreType.REGULAR,
            pltpu.SemaphoreType.REGULAR,
        ],
        compiler_params=pltpu.CompilerParams(
            collective_id=0,
            dimension_semantics=("arbitrary",),
        ),
    )(x, w_mat)


# device time: 315399 ns/iter; 1.0024x vs baseline; 1.0024x over previous
import jax
import jax.numpy as jnp
from jax import lax
from jax.experimental import pallas as pl
from jax.experimental.pallas import tpu as pltpu

N_DEV = 4
NB = 8


def kernel(x, w_mat):
    m, k = x.shape
    _, n = w_mat.shape
    m_out = m // N_DEV
    nb = n // NB
    nh = nb // 2
    nq = nh // 2
    f32 = jnp.float32
    bf16 = jnp.bfloat16

    def body(
        x_ref, w_ref, out_ref,
        own_part, bd_part, send_cw, send_ccw, recv_cw, recv_ccw,
        send_sems_cw, send_sems_ccw, recv_sems_cw, recv_sems_ccw,
        credit_cw, credit_ccw,
    ):
        j = pl.program_id(0)
        my = lax.axis_index("i")
        left = lax.rem(my + (N_DEV - 1), N_DEV)
        right = lax.rem(my + 1, N_DEV)

        cw = (send_cw, recv_cw, send_sems_cw, recv_sems_cw,
              credit_cw, left, right)
        ccw = (send_ccw, recv_ccw, send_sems_ccw, recv_sems_ccw,
               credit_ccw, right, left)

        def piece_rdma(d, s, p, target):
            sbuf, rbuf, ssems, rsems = d[0], d[1], d[2], d[3]
            return pltpu.make_async_remote_copy(
                src_ref=sbuf.at[p],
                dst_ref=rbuf.at[s, p],
                send_sem=ssems.at[p],
                recv_sem=rsems.at[s, p],
                device_id=(target,),
                device_id_type=pl.DeviceIdType.MESH,
            )

        def xslice(off):
            c = lax.rem(my + off, N_DEV)
            return x_ref[pl.ds(c * m_out, m_out), :].astype(bf16)

        def w_lo():
            return w_ref[:, :nh].astype(bf16)

        def w_hi():
            return w_ref[:, nh:].astype(bf16)

        def wait_send(d, p):
            piece_rdma(d, 0, p, d[6]).wait_send()

        def wait_all_sends():
            for d in (cw, ccw):
                for p in (0, 1):
                    wait_send(d, p)

        def wait_credits():
            pl.semaphore_wait(credit_cw, 2)
            pl.semaphore_wait(credit_ccw, 2)

        def start_piece(d, s, p):
            piece_rdma(d, s, p, d[6]).start()

        def recv_piece(d, s, p):
            piece_rdma(d, s, p, d[5]).wait_recv()

        def grant_credits(inc):
            pl.semaphore_signal(
                credit_cw, inc=inc,
                device_id=(left,), device_id_type=pl.DeviceIdType.MESH,
            )
            pl.semaphore_signal(
                credit_ccw, inc=inc,
                device_id=(right,), device_id_type=pl.DeviceIdType.MESH,
            )

        def hop(s_recv, s_send):
            for p in (0, 1):
                for d, off in ((cw, 0), (ccw, nh)):
                    lo, hi = off + p * nq, off + (p + 1) * nq
                    recv_piece(d, s_recv, p)
                    acc = (bd_part[:, lo:hi].astype(f32)
                           + d[1][s_recv, p].astype(f32))
                    wait_send(d, p)
                    d[0][p] = acc.astype(bf16)
                    start_piece(d, s_send, p)

        def finalize():
            for p in (0, 1):
                recv_piece(cw, 2, p)
                recv_piece(ccw, 2, p)
            for p in (0, 1):
                lo, hi = p * nq, (p + 1) * nq
                out_ref[:, lo:hi] = jnp.maximum(
                    own_part[:, lo:hi].astype(f32) + recv_cw[2, p].astype(f32),
                    0.0)
                out_ref[:, nh + lo:nh + hi] = jnp.maximum(
                    own_part[:, nh + lo:nh + hi].astype(f32)
                    + recv_ccw[2, p].astype(f32), 0.0)

        @pl.when(j == 0)
        def _():
            barrier_sem = pltpu.get_barrier_semaphore()
            for nbr in (left, right):
                pl.semaphore_signal(
                    barrier_sem, inc=1,
                    device_id=(nbr,), device_id_type=pl.DeviceIdType.MESH,
                )
            pl.semaphore_wait(barrier_sem, 2)

        @pl.when(j < NB)
        def _():
            a_cw = jnp.dot(xslice(3), w_lo(), preferred_element_type=f32)
            a_ccw = jnp.dot(xslice(1), w_hi(), preferred_element_type=f32)

            @pl.when(j > 0)
            def _():
                wait_all_sends()
                wait_credits()
            for p in (0, 1):
                send_cw[p] = a_cw[:, p * nq:(p + 1) * nq].astype(bf16)
                send_ccw[p] = a_ccw[:, p * nq:(p + 1) * nq].astype(bf16)
                start_piece(cw, 0, p)
                start_piece(ccw, 0, p)

            bd_part[:, :nh] = jnp.dot(
                xslice(2), w_lo(), preferred_element_type=f32).astype(bf16)
            bd_part[:, nh:] = jnp.dot(
                xslice(2), w_hi(), preferred_element_type=f32).astype(bf16)

            @pl.when(j > 0)
            def _():
                finalize()
                grant_credits(2)

            @pl.when(j > 0)
            def _():
                wait_credits()
            hop(0, 1)

            @pl.when(j < NB - 1)
            def _():
                grant_credits(2)

            bd_part[:, :nh] = jnp.dot(
                xslice(1), w_lo(), preferred_element_type=f32).astype(bf16)
            bd_part[:, nh:] = jnp.dot(
                xslice(3), w_hi(), preferred_element_type=f32).astype(bf16)

            own_part[:, :nh] = jnp.dot(
                xslice(0), w_lo(), preferred_element_type=f32).astype(bf16)
            own_part[:, nh:] = jnp.dot(
                xslice(0), w_hi(), preferred_element_type=f32).astype(bf16)

            @pl.when(j > 0)
            def _():
                wait_credits()
            hop(1, 2)

            @pl.when(j < NB - 1)
            def _():
                grant_credits(2)

        @pl.when(j == NB)
        def _():
            finalize()
            wait_all_sends()

    return pl.pallas_call(
        body,
        grid=(NB + 1,),
        out_shape=jax.ShapeDtypeStruct((m_out, n), f32),
        in_specs=[
            pl.BlockSpec((m, k), lambda j: (0, 0)),
            pl.BlockSpec((k, nb), lambda j: (0, jnp.minimum(j, NB - 1))),
        ],
        out_specs=pl.BlockSpec((m_out, nb), lambda j: (0, jnp.maximum(j - 1, 0))),
        scratch_shapes=[
            pltpu.VMEM((m_out, nb), bf16),
            pltpu.VMEM((m_out, nb), bf16),
            pltpu.VMEM((2, m_out, nq), bf16),
            pltpu.VMEM((2, m_out, nq), bf16),
            pltpu.VMEM((N_DEV - 1, 2, m_out, nq), bf16),
            pltpu.VMEM((N_DEV - 1, 2, m_out, nq), bf16),
            pltpu.SemaphoreType.DMA((2,)),
            pltpu.SemaphoreType.DMA((2,)),
            pltpu.SemaphoreType.DMA((N_DEV - 1, 2)),
            pltpu.SemaphoreType.DMA((N_DEV - 1, 2)),
            pltpu.SemaphoreType.REGULAR,
            pltpu.SemaphoreType.REGULAR,
        ],
        compiler_params=pltpu.CompilerParams(
            collective_id=0,
            dimension_semantics=("arbitrary",),
        ),
    )(x, w_mat)


# device time: 310742 ns/iter; 1.0174x vs baseline; 1.0150x over previous
import jax
import jax.numpy as jnp
from jax import lax
from jax.experimental import pallas as pl
from jax.experimental.pallas import tpu as pltpu

N_DEV = 4
NB = 8


def kernel(x, w_mat):
    m, k = x.shape
    _, n = w_mat.shape
    m_out = m // N_DEV
    nb = n // NB
    nh = nb // 2
    nq = nh // 2
    f32 = jnp.float32
    bf16 = jnp.bfloat16

    def body(
        x_ref, w_ref, out_ref,
        own_part, bd_part, send_cw, send_ccw, recv_cw, recv_ccw,
        send_sems_cw, send_sems_ccw, recv_sems_cw, recv_sems_ccw,
        credit_cw, credit_ccw,
    ):
        j = pl.program_id(0)
        my = lax.axis_index("i")
        left = lax.rem(my + (N_DEV - 1), N_DEV)
        right = lax.rem(my + 1, N_DEV)

        cw = (send_cw, recv_cw, send_sems_cw, recv_sems_cw,
              credit_cw, left, right)
        ccw = (send_ccw, recv_ccw, send_sems_ccw, recv_sems_ccw,
               credit_ccw, right, left)

        def piece_rdma(d, s, p, target):
            b = 0 if s == 0 else 1
            sbuf, rbuf, ssems, rsems = d[0], d[1], d[2], d[3]
            return pltpu.make_async_remote_copy(
                src_ref=sbuf.at[b, p],
                dst_ref=rbuf.at[s, p],
                send_sem=ssems.at[b, p],
                recv_sem=rsems.at[s, p],
                device_id=(target,),
                device_id_type=pl.DeviceIdType.MESH,
            )

        def xslice(off):
            c = lax.rem(my + off, N_DEV)
            return x_ref[pl.ds(c * m_out, m_out), :].astype(bf16)

        def w_lo():
            return w_ref[:, :nh].astype(bf16)

        def w_hi():
            return w_ref[:, nh:].astype(bf16)

        def wait_send(d, s, p):
            piece_rdma(d, s, p, d[6]).wait_send()

        def wait_all_sends():
            for d in (cw, ccw):
                for s in (0, 2):
                    for p in (0, 1):
                        wait_send(d, s, p)

        def wait_credits():
            pl.semaphore_wait(credit_cw, 2)
            pl.semaphore_wait(credit_ccw, 2)

        def start_piece(d, s, p):
            piece_rdma(d, s, p, d[6]).start()

        def recv_piece(d, s, p):
            piece_rdma(d, s, p, d[5]).wait_recv()

        def grant_credits(inc):
            pl.semaphore_signal(
                credit_cw, inc=inc,
                device_id=(left,), device_id_type=pl.DeviceIdType.MESH,
            )
            pl.semaphore_signal(
                credit_ccw, inc=inc,
                device_id=(right,), device_id_type=pl.DeviceIdType.MESH,
            )

        def hop(s_recv, s_send):
            for p in (0, 1):
                for d, off in ((cw, 0), (ccw, nh)):
                    lo, hi = off + p * nq, off + (p + 1) * nq
                    recv_piece(d, s_recv, p)
                    acc = (bd_part[:, lo:hi].astype(f32)
                           + d[1][s_recv, p].astype(f32))
                    if s_send == 1:
                        @pl.when(j > 0)
                        def _():
                            wait_send(d, s_send, p)
                    else:
                        wait_send(d, s_send, p)
                    d[0][1, p] = acc.astype(bf16)
                    start_piece(d, s_send, p)

        def finalize():
            for p in (0, 1):
                recv_piece(cw, 2, p)
                recv_piece(ccw, 2, p)
            for p in (0, 1):
                lo, hi = p * nq, (p + 1) * nq
                out_ref[:, lo:hi] = jnp.maximum(
                    own_part[:, lo:hi].astype(f32) + recv_cw[2, p].astype(f32),
                    0.0)
                out_ref[:, nh + lo:nh + hi] = jnp.maximum(
                    own_part[:, nh + lo:nh + hi].astype(f32)
                    + recv_ccw[2, p].astype(f32), 0.0)

        @pl.when(j == 0)
        def _():
            barrier_sem = pltpu.get_barrier_semaphore()
            for nbr in (left, right):
                pl.semaphore_signal(
                    barrier_sem, inc=1,
                    device_id=(nbr,), device_id_type=pl.DeviceIdType.MESH,
                )
            pl.semaphore_wait(barrier_sem, 2)

        @pl.when(j < NB)
        def _():
            @pl.when(j > 0)
            def _():
                for d in (cw, ccw):
                    for p in (0, 1):
                        wait_send(d, 0, p)
                wait_credits()
            for p in (0, 1):
                lo, hi = p * nq, (p + 1) * nq
                send_cw[0, p] = jnp.dot(
                    xslice(3), w_ref[:, lo:hi].astype(bf16),
                    preferred_element_type=f32).astype(bf16)
                start_piece(cw, 0, p)
                send_ccw[0, p] = jnp.dot(
                    xslice(1), w_ref[:, nh + lo:nh + hi].astype(bf16),
                    preferred_element_type=f32).astype(bf16)
                start_piece(ccw, 0, p)

            for q in range(4):
                bd_part[:, q * nq:(q + 1) * nq] = jnp.dot(
                    xslice(2), w_ref[:, q * nq:(q + 1) * nq].astype(bf16),
                    preferred_element_type=f32).astype(bf16)

            @pl.when(j > 0)
            def _():
                finalize()
                grant_credits(2)

            @pl.when(j > 0)
            def _():
                wait_credits()
            hop(0, 1)

            @pl.when(j < NB - 1)
            def _():
                grant_credits(2)

            for q in range(4):
                bd_part[:, q * nq:(q + 1) * nq] = jnp.dot(
                    xslice(1 if q < 2 else 3),
                    w_ref[:, q * nq:(q + 1) * nq].astype(bf16),
                    preferred_element_type=f32).astype(bf16)

            for q in range(4):
                own_part[:, q * nq:(q + 1) * nq] = jnp.dot(
                    xslice(0), w_ref[:, q * nq:(q + 1) * nq].astype(bf16),
                    preferred_element_type=f32).astype(bf16)

            @pl.when(j > 0)
            def _():
                wait_credits()
            hop(1, 2)

            @pl.when(j < NB - 1)
            def _():
                grant_credits(2)

        @pl.when(j == NB)
        def _():
            finalize()
            wait_all_sends()

    return pl.pallas_call(
        body,
        grid=(NB + 1,),
        out_shape=jax.ShapeDtypeStruct((m_out, n), f32),
        in_specs=[
            pl.BlockSpec((m, k), lambda j: (0, 0)),
            pl.BlockSpec((k, nb), lambda j: (0, jnp.minimum(j, NB - 1))),
        ],
        out_specs=pl.BlockSpec((m_out, nb), lambda j: (0, jnp.maximum(j - 1, 0))),
        scratch_shapes=[
            pltpu.VMEM((m_out, nb), bf16),
            pltpu.VMEM((m_out, nb), bf16),
            pltpu.VMEM((2, 2, m_out, nq), bf16),
            pltpu.VMEM((2, 2, m_out, nq), bf16),
            pltpu.VMEM((N_DEV - 1, 2, m_out, nq), bf16),
            pltpu.VMEM((N_DEV - 1, 2, m_out, nq), bf16),
            pltpu.SemaphoreType.DMA((2, 2)),
            pltpu.SemaphoreType.DMA((2, 2)),
            pltpu.SemaphoreType.DMA((N_DEV - 1, 2)),
            pltpu.SemaphoreType.DMA((N_DEV - 1, 2)),
            pltpu.SemaphoreType.REGULAR,
            pltpu.SemaphoreType.REGULAR,
        ],
        compiler_params=pltpu.CompilerParams(
            collective_id=0,
            dimension_semantics=("arbitrary",),
        ),
    )(x, w_mat)
